# Initial kernel scaffold; baseline (speedup 1.0000x reference)
import jax
import jax.numpy as jnp
from jax.experimental import pallas as pl


def kernel(x_node, x_edge, edge_index, W, b):
    raise NotImplementedError("write your pallas kernel here")


# speedup vs baseline: 4.1904x; 4.1904x over previous
import functools

import jax
import jax.numpy as jnp
from jax import lax
from jax.experimental import pallas as pl
from jax.experimental.pallas import tpu as pltpu
from jax.experimental.pallas import tpu_sc as plsc

N_NODES = 10000
N_EDGES = 320000
D_FEAT = 128
D_EDGE = 16

NC, NS = 2, 16
NW = NC * NS
EPW = N_EDGES // NW
CHUNK = 1000
NCHUNK = EPW // CHUNK


def _node_proj_body(x_ref, w_ref, psrc_ref, pdst_ref):
    p = jnp.dot(x_ref[...], w_ref[...], preferred_element_type=jnp.float32)
    psrc_ref[...] = p[:, :D_EDGE]
    pdst_ref[...] = p[:, D_EDGE:]


def _node_proj(x_node, w_sd):
    return pl.pallas_call(
        _node_proj_body,
        out_shape=(
            jax.ShapeDtypeStruct((N_NODES, D_EDGE), jnp.float32),
            jax.ShapeDtypeStruct((N_NODES, D_EDGE), jnp.float32),
        ),
    )(x_node, w_sd)


_EBLK = 5000


def _edge_term_body(x_ref, w_ref, b_ref, o_ref):
    o_ref[...] = (
        jnp.dot(x_ref[...], w_ref[...], preferred_element_type=jnp.float32)
        + b_ref[...]
    )


def _edge_term(x_e128, w_big, b_big):
    n_rows = x_e128.shape[0]
    return pl.pallas_call(
        _edge_term_body,
        grid=(n_rows // _EBLK,),
        in_specs=[
            pl.BlockSpec((_EBLK, 128), lambda i: (i, 0)),
            pl.BlockSpec((128, 128), lambda i: (0, 0)),
            pl.BlockSpec((1, 128), lambda i: (0, 0)),
        ],
        out_specs=pl.BlockSpec((_EBLK, 128), lambda i: (i, 0)),
        out_shape=jax.ShapeDtypeStruct((n_rows, 128), jnp.float32),
    )(x_e128, w_big, b_big)


def _sc_body(psrc_hbm, pdst_hbm, eterm_hbm, e0_hbm, e1_hbm, out_hbm,
             idx0_v, idx1_v, s_v, d_v, t_v, sem0, sem1):
    cid = lax.axis_index("c")
    sid = lax.axis_index("s")
    wid = sid * NC + cid
    wbase = wid * EPW

    def chunk(i, carry):
        base = wbase + i * CHUNK
        pltpu.sync_copy(e0_hbm.at[pl.ds(base, CHUNK)], idx0_v)
        pltpu.sync_copy(e1_hbm.at[pl.ds(base, CHUNK)], idx1_v)
        cp_s = pltpu.async_copy(psrc_hbm.at[idx0_v], s_v, sem0)
        cp_d = pltpu.async_copy(pdst_hbm.at[idx1_v], d_v, sem1)
        pltpu.sync_copy(eterm_hbm.at[pl.ds(base, CHUNK)], t_v)
        cp_s.wait()
        cp_d.wait()

        def row(r, c):
            t_v[r, :] = t_v[r, :] + s_v[r, :] + d_v[r, :]
            return c

        lax.fori_loop(0, CHUNK, row, 0)
        pltpu.sync_copy(t_v, out_hbm.at[pl.ds(base, CHUNK)])
        return carry

    lax.fori_loop(0, NCHUNK, chunk, 0)


@functools.partial(
    pl.kernel,
    out_type=jax.ShapeDtypeStruct((N_EDGES, D_EDGE), jnp.float32),
    mesh=plsc.VectorSubcoreMesh(core_axis_name="c", subcore_axis_name="s"),
    compiler_params=pltpu.CompilerParams(use_tc_tiling_on_sc=False),
    scratch_types=[
        pltpu.VMEM((CHUNK,), jnp.int32),
        pltpu.VMEM((CHUNK,), jnp.int32),
        pltpu.VMEM((CHUNK, D_EDGE), jnp.float32),
        pltpu.VMEM((CHUNK, D_EDGE), jnp.float32),
        pltpu.VMEM((CHUNK, D_EDGE), jnp.float32),
        pltpu.SemaphoreType.DMA,
        pltpu.SemaphoreType.DMA,
    ],
)
def _sc_gather_sum(psrc, pdst, eterm, e0, e1, out,
                   idx0_v, idx1_v, s_v, d_v, t_v, sem0, sem1):
    _sc_body(psrc, pdst, eterm, e0, e1, out,
             idx0_v, idx1_v, s_v, d_v, t_v, sem0, sem1)


def kernel(x_node, x_edge, edge_index, W, b):
    w_sd = jnp.concatenate([W[:D_FEAT], W[D_FEAT:2 * D_FEAT]], axis=1)
    w_big = jnp.kron(jnp.eye(8, dtype=W.dtype), W[2 * D_FEAT:])
    b_big = jnp.tile(b, 8)[None, :]

    psrc, pdst = _node_proj(x_node, w_sd)
    eterm = _edge_term(x_edge.reshape(N_EDGES // 8, 128), w_big, b_big)
    eterm = eterm.reshape(N_EDGES, D_EDGE)

    e0 = edge_index[0]
    e1 = edge_index[1]
    x_edge_new = _sc_gather_sum(psrc, pdst, eterm, e0, e1)
    return (x_edge_new, x_node, edge_index)


# speedup vs baseline: 5.2284x; 1.2477x over previous
import functools

import jax
import jax.numpy as jnp
from jax import lax
from jax.experimental import pallas as pl
from jax.experimental.pallas import tpu as pltpu
from jax.experimental.pallas import tpu_sc as plsc

N_NODES = 10000
N_EDGES = 320000
D_FEAT = 128
D_EDGE = 16

NC, NS = 2, 16
NW = NC * NS
EPW = N_EDGES // NW
CHUNK = 1000
NCHUNK = EPW // CHUNK
CROWS = CHUNK // 8


def _node_proj_body(x_ref, w_ref, psrc_ref, pdst_ref):
    p = jnp.dot(x_ref[...], w_ref[...], preferred_element_type=jnp.float32)
    psrc_ref[...] = p[:, :D_EDGE]
    pdst_ref[...] = p[:, D_EDGE:]


def _node_proj(x_node, w_sd):
    return pl.pallas_call(
        _node_proj_body,
        out_shape=(
            jax.ShapeDtypeStruct((N_NODES, D_EDGE), jnp.float32),
            jax.ShapeDtypeStruct((N_NODES, D_EDGE), jnp.float32),
        ),
    )(x_node, w_sd)


_EBLK = 5000


def _combine_body(x_ref, g_ref, w_ref, b_ref, o_ref):
    o_ref[...] = (
        jnp.dot(x_ref[...], w_ref[...], preferred_element_type=jnp.float32)
        + b_ref[...]
        + g_ref[...]
    )


def _combine(x_e128, gsum, w_big, b_big):
    n_rows = x_e128.shape[0]
    return pl.pallas_call(
        _combine_body,
        grid=(n_rows // _EBLK,),
        in_specs=[
            pl.BlockSpec((_EBLK, 128), lambda i: (i, 0)),
            pl.BlockSpec((_EBLK, 128), lambda i: (i, 0)),
            pl.BlockSpec((128, 128), lambda i: (0, 0)),
            pl.BlockSpec((1, 128), lambda i: (0, 0)),
        ],
        out_specs=pl.BlockSpec((_EBLK, 128), lambda i: (i, 0)),
        out_shape=jax.ShapeDtypeStruct((n_rows, 128), jnp.float32),
    )(x_e128, gsum, w_big, b_big)


def _sc_body(psrc_hbm, pdst_hbm, e0_hbm, e1_hbm, out_hbm,
             idx0_v, idx1_v, s_v, d_v, t_v, sem0, sem1):
    cid = lax.axis_index("c")
    sid = lax.axis_index("s")
    wid = sid * NC + cid
    wbase = wid * EPW

    def chunk(i, carry):
        base = wbase + i * CHUNK
        pltpu.sync_copy(e0_hbm.at[pl.ds(base, CHUNK)], idx0_v)
        pltpu.sync_copy(e1_hbm.at[pl.ds(base, CHUNK)], idx1_v)
        cp_s = pltpu.async_copy(psrc_hbm.at[idx0_v], s_v, sem0)
        cp_d = pltpu.async_copy(pdst_hbm.at[idx1_v], d_v, sem1)
        cp_s.wait()
        cp_d.wait()

        def row8(r8, c):
            for k in range(8):
                r = r8 * 8 + k
                t_v[r8, pl.ds(k * D_EDGE, D_EDGE)] = s_v[r, :] + d_v[r, :]
            return c

        lax.fori_loop(0, CROWS, row8, 0)
        pltpu.sync_copy(t_v, out_hbm.at[pl.ds(base // 8, CROWS)])
        return carry

    lax.fori_loop(0, NCHUNK, chunk, 0)


@functools.partial(
    pl.kernel,
    out_type=jax.ShapeDtypeStruct((N_EDGES // 8, 128), jnp.float32),
    mesh=plsc.VectorSubcoreMesh(core_axis_name="c", subcore_axis_name="s"),
    compiler_params=pltpu.CompilerParams(use_tc_tiling_on_sc=False),
    scratch_types=[
        pltpu.VMEM((CHUNK,), jnp.int32),
        pltpu.VMEM((CHUNK,), jnp.int32),
        pltpu.VMEM((CHUNK, D_EDGE), jnp.float32),
        pltpu.VMEM((CHUNK, D_EDGE), jnp.float32),
        pltpu.VMEM((CROWS, 128), jnp.float32),
        pltpu.SemaphoreType.DMA,
        pltpu.SemaphoreType.DMA,
    ],
)
def _sc_gather_sum(psrc, pdst, e0, e1, out,
                   idx0_v, idx1_v, s_v, d_v, t_v, sem0, sem1):
    _sc_body(psrc, pdst, e0, e1, out,
             idx0_v, idx1_v, s_v, d_v, t_v, sem0, sem1)


def kernel(x_node, x_edge, edge_index, W, b):
    w_sd = jnp.concatenate([W[:D_FEAT], W[D_FEAT:2 * D_FEAT]], axis=1)
    w_big = jnp.kron(jnp.eye(8, dtype=W.dtype), W[2 * D_FEAT:])
    b_big = jnp.tile(b, 8)[None, :]

    psrc, pdst = _node_proj(x_node, w_sd)
    e0 = edge_index[0]
    e1 = edge_index[1]
    gsum = _sc_gather_sum(psrc, pdst, e0, e1)
    out = _combine(x_edge.reshape(N_EDGES // 8, 128), gsum, w_big, b_big)
    return (out.reshape(N_EDGES, D_EDGE), x_node, edge_index)


# speedup vs baseline: 6.8243x; 1.3053x over previous
import functools

import jax
import jax.numpy as jnp
from jax import lax
from jax.experimental import pallas as pl
from jax.experimental.pallas import tpu as pltpu
from jax.experimental.pallas import tpu_sc as plsc

N_NODES = 10000
N_EDGES = 320000
D_FEAT = 128
D_EDGE = 16

NC, NS = 2, 16
NW = NC * NS
EPW = N_EDGES // NW
CHUNK = 1000
NCHUNK = EPW // CHUNK
CROWS = CHUNK // 8


def _node_proj_body(x_ref, w_ref, b_ref, ei_ref,
                    psrc_ref, pdst_ref, xc_ref, ec_ref):
    p = jnp.dot(x_ref[...], w_ref[...], preferred_element_type=jnp.float32)
    psrc_ref[...] = p[:, :D_EDGE] + b_ref[...]
    pdst_ref[...] = p[:, D_EDGE:]
    xc_ref[...] = x_ref[...]
    ec_ref[...] = ei_ref[...]


def _node_proj(x_node, w_sd, b_row, edge_index):
    return pl.pallas_call(
        _node_proj_body,
        out_shape=(
            jax.ShapeDtypeStruct((N_NODES, D_EDGE), jnp.float32),
            jax.ShapeDtypeStruct((N_NODES, D_EDGE), jnp.float32),
            jax.ShapeDtypeStruct((N_NODES, D_FEAT), jnp.float32),
            jax.ShapeDtypeStruct((2, N_EDGES), jnp.int32),
        ),
    )(x_node, w_sd, b_row, edge_index)


_EBLK = 16000


def _combine_body(xt_ref, g_ref, wt_ref, eye_ref, o_ref):
    xw = jnp.dot(wt_ref[...], xt_ref[...], preferred_element_type=jnp.float32)
    gt = lax.dot_general(
        eye_ref[...], g_ref[...],
        dimension_numbers=(((1,), (1,)), ((), ())),
        preferred_element_type=jnp.float32,
    )
    o_ref[...] = xw + gt


def _combine(x_t, g2, w_t, eye16):
    return pl.pallas_call(
        _combine_body,
        grid=(N_EDGES // _EBLK,),
        in_specs=[
            pl.BlockSpec((D_EDGE, _EBLK), lambda i: (0, i)),
            pl.BlockSpec((_EBLK, D_EDGE), lambda i: (i, 0)),
            pl.BlockSpec((D_EDGE, D_EDGE), lambda i: (0, 0)),
            pl.BlockSpec((D_EDGE, D_EDGE), lambda i: (0, 0)),
        ],
        out_specs=pl.BlockSpec((D_EDGE, _EBLK), lambda i: (0, i)),
        out_shape=jax.ShapeDtypeStruct((D_EDGE, N_EDGES), jnp.float32),
    )(x_t, g2, w_t, eye16)


def _sc_body(psrc_hbm, pdst_hbm, e0_hbm, e1_hbm, out_hbm,
             idx0_v, idx1_v, s_v, d_v, t_v, sem0, sem1):
    cid = lax.axis_index("c")
    sid = lax.axis_index("s")
    wid = sid * NC + cid
    wbase = wid * EPW

    def chunk(i, carry):
        base = wbase + i * CHUNK
        pltpu.sync_copy(e0_hbm.at[pl.ds(base, CHUNK)], idx0_v)
        pltpu.sync_copy(e1_hbm.at[pl.ds(base, CHUNK)], idx1_v)
        cp_s = pltpu.async_copy(psrc_hbm.at[idx0_v], s_v, sem0)
        cp_d = pltpu.async_copy(pdst_hbm.at[idx1_v], d_v, sem1)
        cp_s.wait()
        cp_d.wait()

        def row8(r8, c):
            for k in range(8):
                r = r8 * 8 + k
                t_v[r8, pl.ds(k * D_EDGE, D_EDGE)] = s_v[r, :] + d_v[r, :]
            return c

        lax.fori_loop(0, CROWS, row8, 0)
        pltpu.sync_copy(t_v, out_hbm.at[pl.ds(base // 8, CROWS)])
        return carry

    lax.fori_loop(0, NCHUNK, chunk, 0)


@functools.partial(
    pl.kernel,
    out_type=jax.ShapeDtypeStruct((N_EDGES // 8, 128), jnp.float32),
    mesh=plsc.VectorSubcoreMesh(core_axis_name="c", subcore_axis_name="s"),
    compiler_params=pltpu.CompilerParams(use_tc_tiling_on_sc=False),
    scratch_types=[
        pltpu.VMEM((CHUNK,), jnp.int32),
        pltpu.VMEM((CHUNK,), jnp.int32),
        pltpu.VMEM((CHUNK, D_EDGE), jnp.float32),
        pltpu.VMEM((CHUNK, D_EDGE), jnp.float32),
        pltpu.VMEM((CROWS, 128), jnp.float32),
        pltpu.SemaphoreType.DMA,
        pltpu.SemaphoreType.DMA,
    ],
)
def _sc_gather_sum(psrc, pdst, e0, e1, out,
                   idx0_v, idx1_v, s_v, d_v, t_v, sem0, sem1):
    _sc_body(psrc, pdst, e0, e1, out,
             idx0_v, idx1_v, s_v, d_v, t_v, sem0, sem1)


def kernel(x_node, x_edge, edge_index, W, b):
    w_sd = jnp.concatenate([W[:D_FEAT], W[D_FEAT:2 * D_FEAT]], axis=1)
    w_t = W[2 * D_FEAT:].T
    b_row = b[None, :]
    eye16 = jnp.eye(D_EDGE, dtype=jnp.float32)

    psrc, pdst, x_node_out, edge_index_out = _node_proj(
        x_node, w_sd, b_row, edge_index)
    e0 = edge_index[0]
    e1 = edge_index[1]
    gsum = _sc_gather_sum(psrc, pdst, e0, e1)
    out_t = _combine(x_edge.T, gsum.reshape(N_EDGES, D_EDGE), w_t, eye16)
    return (out_t.T, x_node_out, edge_index_out)


# speedup vs baseline: 9.5365x; 1.3974x over previous
import functools

import jax
import jax.numpy as jnp
from jax import lax
from jax.experimental import pallas as pl
from jax.experimental.pallas import tpu as pltpu
from jax.experimental.pallas import tpu_sc as plsc

N_NODES = 10000
N_EDGES = 320000
D_FEAT = 128
D_EDGE = 16

NC, NS = 2, 16
NW = NC * NS
EPW = N_EDGES // NW
CHUNK = 1000
NCHUNK = EPW // CHUNK

_EBLK = 16000
_NBLK = N_EDGES // _EBLK


def _node_proj_body(x8_ref, ws_ref, wd_ref, bt_ref, ei_ref,
                    psrc_ref, pdst_ref, xc_ref, ec_ref):
    x8 = x8_ref[...]
    psrc_ref[...] = (
        jnp.dot(x8, ws_ref[...], preferred_element_type=jnp.float32)
        + bt_ref[...]
    )
    pdst_ref[...] = jnp.dot(x8, wd_ref[...], preferred_element_type=jnp.float32)
    xc_ref[...] = x8
    ec_ref[...] = ei_ref[...]


def _node_proj(x8, ws_blk, wd_blk, b_tile, edge_index):
    return pl.pallas_call(
        _node_proj_body,
        out_shape=(
            jax.ShapeDtypeStruct((N_NODES // 8, 128), jnp.float32),
            jax.ShapeDtypeStruct((N_NODES // 8, 128), jnp.float32),
            jax.ShapeDtypeStruct((N_NODES // 8, 8 * D_FEAT), jnp.float32),
            jax.ShapeDtypeStruct((2, N_EDGES), jnp.int32),
        ),
    )(x8, ws_blk, wd_blk, b_tile, edge_index)


def _combine_body(xt_ref, g_ref, wt_ref, o_ref):
    o_ref[...] = (
        jnp.dot(wt_ref[...], xt_ref[...], preferred_element_type=jnp.float32)
        + g_ref[0]
    )


def _combine(x_t, gsum3, w_t):
    return pl.pallas_call(
        _combine_body,
        grid=(_NBLK,),
        in_specs=[
            pl.BlockSpec((D_EDGE, _EBLK), lambda i: (0, i)),
            pl.BlockSpec((1, D_EDGE, _EBLK), lambda i: (i, 0, 0)),
            pl.BlockSpec((D_EDGE, D_EDGE), lambda i: (0, 0)),
        ],
        out_specs=pl.BlockSpec((D_EDGE, _EBLK), lambda i: (0, i)),
        out_shape=jax.ShapeDtypeStruct((D_EDGE, N_EDGES), jnp.float32),
    )(x_t, gsum3, w_t)


def _sc_body(psrc_hbm, pdst_hbm, e0_hbm, e1_hbm, out_hbm,
             idx0_v, idx1_v, s_v, d_v, t_v, sem0, sem1):
    cid = lax.axis_index("c")
    sid = lax.axis_index("s")
    wid = sid * NC + cid
    wbase = wid * EPW
    iota16 = lax.iota(jnp.int32, 16)

    def chunk(i, carry):
        base = wbase + i * CHUNK
        blk = base // _EBLK
        col = base % _EBLK
        pltpu.sync_copy(e0_hbm.at[pl.ds(base, CHUNK)], idx0_v)
        pltpu.sync_copy(e1_hbm.at[pl.ds(base, CHUNK)], idx1_v)
        cp_s = pltpu.async_copy(psrc_hbm.at[idx0_v], s_v, sem0)
        cp_d = pltpu.async_copy(pdst_hbm.at[idx1_v], d_v, sem1)
        cp_s.wait()
        cp_d.wait()

        def row(e, c):
            v = s_v[e, :] + d_v[e, :]
            plsc.store_scatter(t_v, [iota16, jnp.full((16,), e, jnp.int32)], v)
            return c

        lax.fori_loop(0, CHUNK, row, 0)
        pltpu.sync_copy(t_v, out_hbm.at[blk, :, pl.ds(col, CHUNK)])
        return carry

    lax.fori_loop(0, NCHUNK, chunk, 0)


@functools.partial(
    pl.kernel,
    out_type=jax.ShapeDtypeStruct((_NBLK, D_EDGE, _EBLK), jnp.float32),
    mesh=plsc.VectorSubcoreMesh(core_axis_name="c", subcore_axis_name="s"),
    compiler_params=pltpu.CompilerParams(
        use_tc_tiling_on_sc=False, needs_layout_passes=False),
    scratch_types=[
        pltpu.VMEM((CHUNK,), jnp.int32),
        pltpu.VMEM((CHUNK,), jnp.int32),
        pltpu.VMEM((CHUNK, D_EDGE), jnp.float32),
        pltpu.VMEM((CHUNK, D_EDGE), jnp.float32),
        pltpu.VMEM((D_EDGE, CHUNK), jnp.float32),
        pltpu.SemaphoreType.DMA,
        pltpu.SemaphoreType.DMA,
    ],
)
def _sc_gather_sum(psrc, pdst, e0, e1, out,
                   idx0_v, idx1_v, s_v, d_v, t_v, sem0, sem1):
    _sc_body(psrc, pdst, e0, e1, out,
             idx0_v, idx1_v, s_v, d_v, t_v, sem0, sem1)


def kernel(x_node, x_edge, edge_index, W, b):
    eye8 = jnp.eye(8, dtype=W.dtype)
    ws_blk = jnp.kron(eye8, W[:D_FEAT])
    wd_blk = jnp.kron(eye8, W[D_FEAT:2 * D_FEAT])
    b_tile = jnp.tile(b, 8)[None, :]
    w_t = W[2 * D_FEAT:].T

    x8 = x_node.reshape(N_NODES // 8, 8 * D_FEAT)
    psrc128, pdst128, xc8, edge_index_out = _node_proj(
        x8, ws_blk, wd_blk, b_tile, edge_index)

    e0 = edge_index[0]
    e1 = edge_index[1]
    gsum3 = _sc_gather_sum(
        psrc128.reshape(N_NODES, D_EDGE),
        pdst128.reshape(N_NODES, D_EDGE),
        e0, e1)
    out_t = _combine(x_edge.T, gsum3, w_t)
    return (out_t.T, xc8.reshape(N_NODES, D_FEAT), edge_index_out)


# speedup vs baseline: 11.6796x; 1.2247x over previous
import functools

import jax
import jax.numpy as jnp
from jax import lax
from jax.experimental import pallas as pl
from jax.experimental.pallas import tpu as pltpu
from jax.experimental.pallas import tpu_sc as plsc

N_NODES = 10000
N_EDGES = 320000
D_FEAT = 128
D_EDGE = 16

NC, NS = 2, 16
NW = NC * NS
EPW = N_EDGES // NW
CHUNK = 1000
NCHUNK = EPW // CHUNK

_EBLK = 16000
_NBLK = N_EDGES // _EBLK


def _node_proj_body(x8_ref, ws_ref, wd_ref, bt_ref, ei_ref,
                    psrc_ref, pdst_ref, xc_ref, ec_ref):
    x8 = x8_ref[...]
    psrc_ref[...] = (
        jnp.dot(x8, ws_ref[...], preferred_element_type=jnp.float32)
        + bt_ref[...]
    )
    pdst_ref[...] = jnp.dot(x8, wd_ref[...], preferred_element_type=jnp.float32)
    xc_ref[...] = x8
    ec_ref[...] = ei_ref[...]


def _node_proj(x8, ws_blk, wd_blk, b_tile, edge_index):
    return pl.pallas_call(
        _node_proj_body,
        out_shape=(
            jax.ShapeDtypeStruct((N_NODES // 8, 128), jnp.float32),
            jax.ShapeDtypeStruct((N_NODES // 8, 128), jnp.float32),
            jax.ShapeDtypeStruct((N_NODES // 8, 8 * D_FEAT), jnp.float32),
            jax.ShapeDtypeStruct((2, N_EDGES), jnp.int32),
        ),
    )(x8, ws_blk, wd_blk, b_tile, edge_index)


def _combine_body(xt_ref, g_ref, wt_ref, o_ref):
    o_ref[...] = (
        jnp.dot(wt_ref[...], xt_ref[...], preferred_element_type=jnp.float32)
        + g_ref[...]
    )


def _combine(x_t, gsum2, w_t):
    return pl.pallas_call(
        _combine_body,
        grid=(_NBLK,),
        in_specs=[
            pl.BlockSpec((D_EDGE, _EBLK), lambda i: (0, i)),
            pl.BlockSpec((D_EDGE, _EBLK), lambda i: (i, 0)),
            pl.BlockSpec((D_EDGE, D_EDGE), lambda i: (0, 0)),
        ],
        out_specs=pl.BlockSpec((D_EDGE, _EBLK), lambda i: (0, i)),
        out_shape=jax.ShapeDtypeStruct((D_EDGE, N_EDGES), jnp.float32),
    )(x_t, gsum2, w_t)


def _sc_body(psrc, pdst, e0, e1, out,
             idx0, idx1, s_a, s_b, d_a, d_b, t_a, t_b,
             gs_a, gs_b, os_a, os_b):
    cid = lax.axis_index("c")
    sid = lax.axis_index("s")
    wid = sid * NC + cid
    wbase = wid * EPW
    iota16 = lax.iota(jnp.int32, 16)

    pltpu.sync_copy(e0.at[pl.ds(wbase, EPW)], idx0)
    pltpu.sync_copy(e1.at[pl.ds(wbase, EPW)], idx1)

    S = (s_a, s_b)
    D = (d_a, d_b)
    T = (t_a, t_b)
    GS = (gs_a, gs_b)
    OS = (os_a, os_b)

    def start_gathers(i):
        p = i % 2
        cs = pltpu.async_copy(
            psrc.at[idx0.at[pl.ds(i * CHUNK, CHUNK)]], S[p], GS[p])
        cd = pltpu.async_copy(
            pdst.at[idx1.at[pl.ds(i * CHUNK, CHUNK)]], D[p], GS[p])
        return cs, cd

    pending = {0: start_gathers(0)}
    out_cp = {}
    for i in range(NCHUNK):
        p = i % 2
        if i + 1 < NCHUNK:
            pending[i + 1] = start_gathers(i + 1)
        cs, cd = pending.pop(i)
        cs.wait()
        cd.wait()
        if i >= 2:
            out_cp.pop(i - 2).wait()

        s_v, d_v, t_v = S[p], D[p], T[p]

        def blk8(r8, c):
            col0 = jnp.full((16,), r8 * 8, jnp.int32)
            for k in range(8):
                e = r8 * 8 + k
                v = s_v[e, :] + d_v[e, :]
                plsc.store_scatter(t_v, [iota16, col0 + k], v)
            return c

        lax.fori_loop(0, CHUNK // 8, blk8, 0)

        base = wbase + i * CHUNK
        blk = base // _EBLK
        col = base % _EBLK
        out_cp[i] = pltpu.async_copy(
            t_v, out.at[pl.ds(blk * D_EDGE, D_EDGE), pl.ds(col, CHUNK)], OS[p])

    out_cp.pop(NCHUNK - 2).wait()
    out_cp.pop(NCHUNK - 1).wait()


@functools.partial(
    pl.kernel,
    out_type=jax.ShapeDtypeStruct((_NBLK * D_EDGE, _EBLK), jnp.float32),
    mesh=plsc.VectorSubcoreMesh(core_axis_name="c", subcore_axis_name="s"),
    compiler_params=pltpu.CompilerParams(
        use_tc_tiling_on_sc=False, needs_layout_passes=False),
    scratch_types=[
        pltpu.VMEM((EPW,), jnp.int32),
        pltpu.VMEM((EPW,), jnp.int32),
        pltpu.VMEM((CHUNK, D_EDGE), jnp.float32),
        pltpu.VMEM((CHUNK, D_EDGE), jnp.float32),
        pltpu.VMEM((CHUNK, D_EDGE), jnp.float32),
        pltpu.VMEM((CHUNK, D_EDGE), jnp.float32),
        pltpu.VMEM((D_EDGE, CHUNK), jnp.float32),
        pltpu.VMEM((D_EDGE, CHUNK), jnp.float32),
        pltpu.SemaphoreType.DMA,
        pltpu.SemaphoreType.DMA,
        pltpu.SemaphoreType.DMA,
        pltpu.SemaphoreType.DMA,
    ],
)
def _sc_gather_sum(psrc, pdst, e0, e1, out,
                   idx0, idx1, s_a, s_b, d_a, d_b, t_a, t_b,
                   gs_a, gs_b, os_a, os_b):
    _sc_body(psrc, pdst, e0, e1, out,
             idx0, idx1, s_a, s_b, d_a, d_b, t_a, t_b,
             gs_a, gs_b, os_a, os_b)


def kernel(x_node, x_edge, edge_index, W, b):
    eye8 = jnp.eye(8, dtype=W.dtype)
    ws_blk = jnp.kron(eye8, W[:D_FEAT])
    wd_blk = jnp.kron(eye8, W[D_FEAT:2 * D_FEAT])
    b_tile = jnp.tile(b, 8)[None, :]
    w_t = W[2 * D_FEAT:].T

    x8 = x_node.reshape(N_NODES // 8, 8 * D_FEAT)
    psrc128, pdst128, xc8, edge_index_out = _node_proj(
        x8, ws_blk, wd_blk, b_tile, edge_index)

    e0 = edge_index[0]
    e1 = edge_index[1]
    gsum2 = _sc_gather_sum(
        psrc128.reshape(N_NODES, D_EDGE),
        pdst128.reshape(N_NODES, D_EDGE),
        e0, e1)
    out_t = _combine(x_edge.T, gsum2, w_t)
    return (out_t.T, xc8.reshape(N_NODES, D_FEAT), edge_index_out)
